# baseline (speedup 1.0000x reference)
import jax
import jax.numpy as jnp
from jax import lax
from jax.experimental import pallas as pl
from jax.experimental.pallas import tpu as pltpu
from jax.experimental.pallas import tpu_sc as plsc

N = 10000
E = 320000
DIM = 128
SLOPE = 0.2

NC = 2
NS = 16
NW = NC * NS
EPW = E // NW
EB = 80
NB = EPW // EB
RPS = 624
RPS_LAST = N - RPS * (NS - 1)


def _tc1_body(h_ref, wt_ref, a8_ref, z_ref, s12_ref, smax_ref):
    i = pl.program_id(0)
    zb = jnp.dot(h_ref[...], wt_ref[...], preferred_element_type=jnp.float32)
    z_ref[...] = zb
    s12 = lax.dot_general(zb, a8_ref[...], (((1,), (1,)), ((), ())),
                          preferred_element_type=jnp.float32)
    s12_ref[...] = s12
    m = jnp.max(s12, axis=0, keepdims=True)

    @pl.when(i == 0)
    def _():
        smax_ref[...] = m

    @pl.when(i > 0)
    def _():
        smax_ref[...] = jnp.maximum(smax_ref[...], m)


def _tc1(h, wt, a8):
    bm = 1000
    return pl.pallas_call(
        _tc1_body,
        grid=(N // bm,),
        in_specs=[
            pl.BlockSpec((bm, DIM), lambda i: (i, 0)),
            pl.BlockSpec((DIM, DIM), lambda i: (0, 0)),
            pl.BlockSpec((8, DIM), lambda i: (0, 0)),
        ],
        out_specs=[
            pl.BlockSpec((bm, DIM), lambda i: (i, 0)),
            pl.BlockSpec((bm, 8), lambda i: (i, 0)),
            pl.BlockSpec((1, 8), lambda i: (0, 0)),
        ],
        out_shape=[
            jax.ShapeDtypeStruct((N, DIM), jnp.float32),
            jax.ShapeDtypeStruct((N, 8), jnp.float32),
            jax.ShapeDtypeStruct((1, 8), jnp.float32),
        ],
    )(h, wt, a8)


def _sc_body(z_hbm, src_hbm, dst_hbm, s1_hbm, s2_hbm, shift_hbm,
             zeros2d_hbm, acc_out, den_out,
             s1_v, s2_v, shift_v,
             srcv0, dstv0, dsc0, exv0, rows0,
             srcv1, dstv1, dsc1, exv1, rows1,
             acc_sp, den_sp,
             isem0, isem1, gsem0, gsem1, ssem0, ssem1):
    c = lax.axis_index("c")
    s = lax.axis_index("s")
    wid = s * NC + c
    ebase = wid * EPW

    @pl.when(s < NS - 1)
    def _():
        pltpu.sync_copy(zeros2d_hbm.at[pl.ds(s * RPS, RPS)],
                        acc_sp.at[pl.ds(s * RPS, RPS)])

    @pl.when(s == NS - 1)
    def _():
        pltpu.sync_copy(zeros2d_hbm.at[pl.ds((NS - 1) * RPS, RPS_LAST)],
                        acc_sp.at[pl.ds((NS - 1) * RPS, RPS_LAST)])

    z16 = jnp.zeros((16,), jnp.float32)

    @pl.when(s == 0)
    def _():
        def zloop(i, cc):
            s1_v[pl.ds(i * 16, 16)] = z16
            return cc

        lax.fori_loop(0, N // 16, zloop, 0)
        pltpu.sync_copy(s1_v, den_sp)

    pltpu.sync_copy(s1_hbm, s1_v)
    pltpu.sync_copy(s2_hbm, s2_v)
    pltpu.sync_copy(shift_hbm, shift_v)
    plsc.subcore_barrier()

    shift_vec = shift_v[...]

    slot0 = (srcv0, dstv0, dsc0, exv0, rows0, isem0, gsem0, ssem0)
    slot1 = (srcv1, dstv1, dsc1, exv1, rows1, isem1, gsem1, ssem1)

    def idx_start(u, slot):
        srcv, dstv, _, _, _, isem, _, _ = slot
        base = ebase + u * EB
        pltpu.async_copy(src_hbm.at[pl.ds(base, EB)], srcv, isem)
        pltpu.async_copy(dst_hbm.at[pl.ds(base, EB)], dstv, isem)

    def idx_wait(u, slot):
        srcv, dstv, _, _, _, isem, _, _ = slot
        base = ebase + u * EB
        pltpu.make_async_copy(src_hbm.at[pl.ds(base, EB)], srcv, isem).wait()
        pltpu.make_async_copy(dst_hbm.at[pl.ds(base, EB)], dstv, isem).wait()

    def gat_start(slot):
        srcv, _, _, _, rows, _, gsem, _ = slot
        pltpu.async_copy(z_hbm.at[srcv], rows, gsem)

    def gat_wait(slot):
        srcv, _, _, _, rows, _, gsem, _ = slot
        pltpu.make_async_copy(z_hbm.at[srcv], rows, gsem).wait()

    def scat_wait(slot):
        _, _, dsc, exv, rows, _, _, ssem = slot
        pltpu.make_async_copy(rows, acc_sp.at[dsc], ssem).wait()
        pltpu.make_async_copy(exv, den_sp.at[dsc], ssem).wait()

    def section(u, slot, other, *, first=False, do_idx=True,
                do_next_gather=True):
        srcv, dstv, dsc, exv, rows, isem, gsem, ssem = slot
        for i in range(EB // 16):
            sl = pl.ds(i * 16, 16)
            sidx = srcv[sl]
            didx = dstv[sl]
            e = plsc.load_gather(s1_v, [sidx]) + plsc.load_gather(s2_v, [didx])
            e = jnp.where(e >= 0, e, SLOPE * e)
            exv[sl] = jnp.exp(e - shift_vec)
            dsc[sl] = didx
        pltpu.async_copy(exv, den_sp.at[dsc], ssem, add=True)
        gat_wait(slot)
        if do_next_gather:
            if not first:
                scat_wait(other)
            idx_wait(u + 1, other)
            gat_start(other)

        @plsc.parallel_loop(0, EB // 16, unroll=2)
        def _(g):
            ex16 = exv[pl.ds(g * 16, 16)]
            for k in range(16):
                b = g * 16 + k
                sv = jnp.full((16,), ex16[k], jnp.float32)
                for j in range(DIM // 16):
                    rows[b, pl.ds(j * 16, 16)] = rows[b, pl.ds(j * 16, 16)] * sv

        if do_idx:
            idx_start(u + 2, slot)
        pltpu.async_copy(rows, acc_sp.at[dsc], ssem, add=True)

    idx_start(0, slot0)
    idx_start(1, slot1)
    idx_wait(0, slot0)
    gat_start(slot0)
    section(0, slot0, slot1, first=True)

    def pair(g, carry):
        section(2 * g + 1, slot1, slot0)
        section(2 * g + 2, slot0, slot1)
        return carry

    lax.fori_loop(0, (NB - 3) // 2, pair, 0)
    section(NB - 2, slot1, slot0, do_idx=False)
    section(NB - 1, slot0, slot1, do_idx=False, do_next_gather=False)
    scat_wait(slot1)
    scat_wait(slot0)
    plsc.subcore_barrier()

    @pl.when(s < NS - 1)
    def _():
        pltpu.sync_copy(acc_sp.at[pl.ds(s * RPS, RPS)],
                        acc_out.at[pl.ds(c * N + s * RPS, RPS)])

    @pl.when(s == NS - 1)
    def _():
        pltpu.sync_copy(acc_sp.at[pl.ds((NS - 1) * RPS, RPS_LAST)],
                        acc_out.at[pl.ds(c * N + (NS - 1) * RPS, RPS_LAST)])

    @pl.when(s == 0)
    def _():
        pltpu.sync_copy(den_sp, s1_v)
        pltpu.sync_copy(s1_v, den_out.at[pl.ds(c * N, N)])


def _sc_aggregate(z, src, dst, s1, s2, shiftv, zeros2d):
    mesh = plsc.VectorSubcoreMesh(core_axis_name="c", subcore_axis_name="s")
    f = pl.kernel(
        _sc_body,
        out_type=[
            jax.ShapeDtypeStruct((NC * N, DIM), jnp.float32),
            jax.ShapeDtypeStruct((NC * N,), jnp.float32),
        ],
        mesh=mesh,
        compiler_params=pltpu.CompilerParams(needs_layout_passes=False),
        scratch_types=(
            [
                pltpu.VMEM((N,), jnp.float32),
                pltpu.VMEM((N,), jnp.float32),
                pltpu.VMEM((16,), jnp.float32),
            ]
            + 2 * [
                pltpu.VMEM((EB,), jnp.int32),
                pltpu.VMEM((EB,), jnp.int32),
                pltpu.VMEM((EB,), jnp.int32),
                pltpu.VMEM((EB,), jnp.float32),
                pltpu.VMEM((EB, DIM), jnp.float32),
            ]
            + [
                pltpu.VMEM_SHARED((N, DIM), jnp.float32),
                pltpu.VMEM_SHARED((N,), jnp.float32),
            ]
            + 6 * [pltpu.SemaphoreType.DMA]
        ),
    )
    return f(z, src, dst, s1, s2, shiftv, zeros2d)


def _tc2_body(a0_ref, a1_ref, den_ref, out_ref):
    d = den_ref[...]
    dsum = jnp.maximum(d[:, 0:1] + d[:, 1:2], 1e-9)
    hout = (a0_ref[...] + a1_ref[...]) / dsum
    out_ref[...] = jnp.where(hout > 0, hout, jnp.exp(hout) - 1.0)


def _tc2(accp, denT):
    bm = 1000
    g = N // bm
    return pl.pallas_call(
        _tc2_body,
        grid=(g,),
        in_specs=[
            pl.BlockSpec((bm, DIM), lambda i: (i, 0)),
            pl.BlockSpec((bm, DIM), lambda i, g=g: (i + g, 0)),
            pl.BlockSpec((bm, 2), lambda i: (i, 0)),
        ],
        out_specs=pl.BlockSpec((bm, DIM), lambda i: (i, 0)),
        out_shape=jax.ShapeDtypeStruct((N, DIM), jnp.float32),
    )(accp, accp, denT)


@jax.jit
def kernel(h, edge_index, W, a):
    wt = W.T
    a8 = jnp.zeros((8, DIM), jnp.float32)
    a8 = a8.at[0].set(a[0, :DIM]).at[1].set(a[0, DIM:])
    z, s12, smax = _tc1(h, wt, a8)
    s1 = s12[:, 0]
    s2 = s12[:, 1]
    shift = jnp.maximum(smax[0, 0] + smax[0, 1], 0.0)
    shiftv = jnp.full((16,), shift, jnp.float32)
    src = edge_index[0]
    dst = edge_index[1]
    zeros2d = jnp.zeros((N, DIM), jnp.float32)
    accp, denp = _sc_aggregate(z, src, dst, s1, s2, shiftv, zeros2d)
    denT = denp.reshape(NC, N).T
    return _tc2(accp, denT)


# speedup vs baseline: 1.0792x; 1.0792x over previous
import jax
import jax.numpy as jnp
from jax import lax
from jax.experimental import pallas as pl
from jax.experimental.pallas import tpu as pltpu
from jax.experimental.pallas import tpu_sc as plsc

N = 10000
E = 320000
DIM = 128
SLOPE = 0.2

NC = 2
NS = 16
NW = NC * NS
EPW = E // NW
EB = 112
NB = EPW // EB
EB_T = EPW - NB * EB
RPS = 624
RPS_LAST = N - RPS * (NS - 1)


def _tc1_body(h_ref, wt_ref, a8_ref, z_ref, s12_ref, smax_ref):
    i = pl.program_id(0)
    zb = jnp.dot(h_ref[...], wt_ref[...], preferred_element_type=jnp.float32)
    z_ref[...] = zb
    s12 = lax.dot_general(zb, a8_ref[...], (((1,), (1,)), ((), ())),
                          preferred_element_type=jnp.float32)
    s12_ref[...] = s12
    m = jnp.max(s12, axis=0, keepdims=True)

    @pl.when(i == 0)
    def _():
        smax_ref[...] = m

    @pl.when(i > 0)
    def _():
        smax_ref[...] = jnp.maximum(smax_ref[...], m)


def _tc1(h, wt, a8):
    bm = 1000
    return pl.pallas_call(
        _tc1_body,
        grid=(N // bm,),
        in_specs=[
            pl.BlockSpec((bm, DIM), lambda i: (i, 0)),
            pl.BlockSpec((DIM, DIM), lambda i: (0, 0)),
            pl.BlockSpec((8, DIM), lambda i: (0, 0)),
        ],
        out_specs=[
            pl.BlockSpec((bm, DIM), lambda i: (i, 0)),
            pl.BlockSpec((bm, 8), lambda i: (i, 0)),
            pl.BlockSpec((1, 8), lambda i: (0, 0)),
        ],
        out_shape=[
            jax.ShapeDtypeStruct((N, DIM), jnp.float32),
            jax.ShapeDtypeStruct((N, 8), jnp.float32),
            jax.ShapeDtypeStruct((1, 8), jnp.float32),
        ],
    )(h, wt, a8)


def _sc_body(z_hbm, src_hbm, dst_hbm, s1_hbm, s2_hbm, shift_hbm,
             zeros2d_hbm, acc_out, den_out,
             s1_v, s2_v, shift_v,
             srcv0, dstv0, dsc0, exv0, rows0,
             srcv1, dstv1, dsc1, exv1, rows1,
             srcv_t, dsc_t, exv_t,
             acc_sp, den_sp,
             isem0, isem1, gsem0, gsem1, ssem0, ssem1):
    c = lax.axis_index("c")
    s = lax.axis_index("s")
    wid = s * NC + c
    ebase = wid * EPW

    @pl.when(s < NS - 1)
    def _():
        pltpu.sync_copy(zeros2d_hbm.at[pl.ds(s * RPS, RPS)],
                        acc_sp.at[pl.ds(s * RPS, RPS)])

    @pl.when(s == NS - 1)
    def _():
        pltpu.sync_copy(zeros2d_hbm.at[pl.ds((NS - 1) * RPS, RPS_LAST)],
                        acc_sp.at[pl.ds((NS - 1) * RPS, RPS_LAST)])

    z16 = jnp.zeros((16,), jnp.float32)

    @pl.when(s == 0)
    def _():
        def zloop(i, cc):
            s1_v[pl.ds(i * 16, 16)] = z16
            return cc

        lax.fori_loop(0, N // 16, zloop, 0)
        pltpu.sync_copy(s1_v, den_sp)

    pltpu.sync_copy(s1_hbm, s1_v)
    pltpu.sync_copy(s2_hbm, s2_v)
    pltpu.sync_copy(shift_hbm, shift_v)
    plsc.subcore_barrier()

    shift_vec = shift_v[...]

    slot0 = (srcv0, dstv0, dsc0, exv0, rows0, isem0, gsem0, ssem0)
    slot1 = (srcv1, dstv1, dsc1, exv1, rows1, isem1, gsem1, ssem1)

    def idx_start(u, slot):
        srcv, dstv, _, _, _, isem, _, _ = slot
        base = ebase + u * EB
        pltpu.async_copy(src_hbm.at[pl.ds(base, EB)], srcv, isem)
        pltpu.async_copy(dst_hbm.at[pl.ds(base, EB)], dstv, isem)

    def idx_wait(u, slot):
        srcv, dstv, _, _, _, isem, _, _ = slot
        base = ebase + u * EB
        pltpu.make_async_copy(src_hbm.at[pl.ds(base, EB)], srcv, isem).wait()
        pltpu.make_async_copy(dst_hbm.at[pl.ds(base, EB)], dstv, isem).wait()

    def gat_start(slot):
        srcv, _, _, _, rows, _, gsem, _ = slot
        pltpu.async_copy(z_hbm.at[srcv], rows, gsem)

    def gat_wait(slot):
        srcv, _, _, _, rows, _, gsem, _ = slot
        pltpu.make_async_copy(z_hbm.at[srcv], rows, gsem).wait()

    def scat_wait(slot):
        _, _, dsc, exv, rows, _, _, ssem = slot
        pltpu.make_async_copy(rows, acc_sp.at[dsc], ssem).wait()
        pltpu.make_async_copy(exv, den_sp.at[dsc], ssem).wait()

    def section(u, slot, other, *, first=False, do_idx=True,
                do_next_gather=True):
        srcv, dstv, dsc, exv, rows, isem, gsem, ssem = slot
        for i in range(EB // 16):
            sl = pl.ds(i * 16, 16)
            sidx = srcv[sl]
            didx = dstv[sl]
            e = plsc.load_gather(s1_v, [sidx]) + plsc.load_gather(s2_v, [didx])
            e = jnp.where(e >= 0, e, SLOPE * e)
            exv[sl] = jnp.exp(e - shift_vec)
            dsc[sl] = didx
        pltpu.async_copy(exv, den_sp.at[dsc], ssem, add=True)
        gat_wait(slot)
        if do_next_gather:
            if not first:
                scat_wait(other)
            idx_wait(u + 1, other)
            gat_start(other)

        @plsc.parallel_loop(0, EB // 16, unroll=2)
        def _(g):
            ex16 = exv[pl.ds(g * 16, 16)]
            for k in range(16):
                b = g * 16 + k
                sv = jnp.full((16,), ex16[k], jnp.float32)
                for j in range(DIM // 16):
                    rows[b, pl.ds(j * 16, 16)] = rows[b, pl.ds(j * 16, 16)] * sv

        if do_idx:
            idx_start(u + 2, slot)
        pltpu.async_copy(rows, acc_sp.at[dsc], ssem, add=True)

    idx_start(0, slot0)
    idx_start(1, slot1)
    idx_wait(0, slot0)
    gat_start(slot0)
    section(0, slot0, slot1, first=True)

    def pair(g, carry):
        section(2 * g + 1, slot1, slot0)
        section(2 * g + 2, slot0, slot1)
        return carry

    lax.fori_loop(0, (NB - 3) // 2, pair, 0)
    section(NB - 2, slot1, slot0, do_idx=False)
    section(NB - 1, slot0, slot1, do_idx=False, do_next_gather=False)
    scat_wait(slot1)
    scat_wait(slot0)

    tbase = ebase + NB * EB
    rows_t = rows0.at[pl.ds(0, EB_T)]
    pltpu.async_copy(src_hbm.at[pl.ds(tbase, EB_T)], srcv_t, isem0)
    pltpu.async_copy(dst_hbm.at[pl.ds(tbase, EB_T)], dsc_t, isem0)
    pltpu.make_async_copy(src_hbm.at[pl.ds(tbase, EB_T)], srcv_t, isem0).wait()
    pltpu.make_async_copy(dst_hbm.at[pl.ds(tbase, EB_T)], dsc_t, isem0).wait()
    pltpu.async_copy(z_hbm.at[srcv_t], rows_t, gsem0)
    for i in range(EB_T // 16):
        sl = pl.ds(i * 16, 16)
        sidx = srcv_t[sl]
        didx = dsc_t[sl]
        e = plsc.load_gather(s1_v, [sidx]) + plsc.load_gather(s2_v, [didx])
        e = jnp.where(e >= 0, e, SLOPE * e)
        exv_t[sl] = jnp.exp(e - shift_vec)
    pltpu.async_copy(exv_t, den_sp.at[dsc_t], ssem0, add=True)
    pltpu.make_async_copy(z_hbm.at[srcv_t], rows_t, gsem0).wait()
    for g in range(EB_T // 16):
        ex16 = exv_t[pl.ds(g * 16, 16)]
        for k in range(16):
            b = g * 16 + k
            sv = jnp.full((16,), ex16[k], jnp.float32)
            for j in range(DIM // 16):
                rows0[b, pl.ds(j * 16, 16)] = rows0[b, pl.ds(j * 16, 16)] * sv
    pltpu.async_copy(rows_t, acc_sp.at[dsc_t], ssem0, add=True)
    pltpu.make_async_copy(exv_t, den_sp.at[dsc_t], ssem0).wait()
    pltpu.make_async_copy(rows_t, acc_sp.at[dsc_t], ssem0).wait()
    plsc.subcore_barrier()

    @pl.when(s < NS - 1)
    def _():
        pltpu.sync_copy(acc_sp.at[pl.ds(s * RPS, RPS)],
                        acc_out.at[pl.ds(c * N + s * RPS, RPS)])

    @pl.when(s == NS - 1)
    def _():
        pltpu.sync_copy(acc_sp.at[pl.ds((NS - 1) * RPS, RPS_LAST)],
                        acc_out.at[pl.ds(c * N + (NS - 1) * RPS, RPS_LAST)])

    @pl.when(s == 0)
    def _():
        pltpu.sync_copy(den_sp, s1_v)
        pltpu.sync_copy(s1_v, den_out.at[pl.ds(c * N, N)])


def _sc_aggregate(z, src, dst, s1, s2, shiftv, zeros2d):
    mesh = plsc.VectorSubcoreMesh(core_axis_name="c", subcore_axis_name="s")
    f = pl.kernel(
        _sc_body,
        out_type=[
            jax.ShapeDtypeStruct((NC * N, DIM), jnp.float32),
            jax.ShapeDtypeStruct((NC * N,), jnp.float32),
        ],
        mesh=mesh,
        compiler_params=pltpu.CompilerParams(needs_layout_passes=False),
        scratch_types=(
            [
                pltpu.VMEM((N,), jnp.float32),
                pltpu.VMEM((N,), jnp.float32),
                pltpu.VMEM((16,), jnp.float32),
            ]
            + 2 * [
                pltpu.VMEM((EB,), jnp.int32),
                pltpu.VMEM((EB,), jnp.int32),
                pltpu.VMEM((EB,), jnp.int32),
                pltpu.VMEM((EB,), jnp.float32),
                pltpu.VMEM((EB, DIM), jnp.float32),
            ]
            + [
                pltpu.VMEM((EB_T,), jnp.int32),
                pltpu.VMEM((EB_T,), jnp.int32),
                pltpu.VMEM((EB_T,), jnp.float32),
            ]
            + [
                pltpu.VMEM_SHARED((N, DIM), jnp.float32),
                pltpu.VMEM_SHARED((N,), jnp.float32),
            ]
            + 6 * [pltpu.SemaphoreType.DMA]
        ),
    )
    return f(z, src, dst, s1, s2, shiftv, zeros2d)


def _tc2_body(a0_ref, a1_ref, den_ref, out_ref):
    d = den_ref[...]
    dsum = jnp.maximum(d[:, 0:1] + d[:, 1:2], 1e-9)
    hout = (a0_ref[...] + a1_ref[...]) / dsum
    out_ref[...] = jnp.where(hout > 0, hout, jnp.exp(hout) - 1.0)


def _tc2(accp, denT):
    bm = 1000
    g = N // bm
    return pl.pallas_call(
        _tc2_body,
        grid=(g,),
        in_specs=[
            pl.BlockSpec((bm, DIM), lambda i: (i, 0)),
            pl.BlockSpec((bm, DIM), lambda i, g=g: (i + g, 0)),
            pl.BlockSpec((bm, 2), lambda i: (i, 0)),
        ],
        out_specs=pl.BlockSpec((bm, DIM), lambda i: (i, 0)),
        out_shape=jax.ShapeDtypeStruct((N, DIM), jnp.float32),
    )(accp, accp, denT)


@jax.jit
def kernel(h, edge_index, W, a):
    wt = W.T
    a8 = jnp.zeros((8, DIM), jnp.float32)
    a8 = a8.at[0].set(a[0, :DIM]).at[1].set(a[0, DIM:])
    z, s12, smax = _tc1(h, wt, a8)
    s1 = s12[:, 0]
    s2 = s12[:, 1]
    shift = jnp.maximum(smax[0, 0] + smax[0, 1], 0.0)
    shiftv = jnp.full((16,), shift, jnp.float32)
    src = edge_index[0]
    dst = edge_index[1]
    zeros2d = jnp.zeros((N, DIM), jnp.float32)
    accp, denp = _sc_aggregate(z, src, dst, s1, s2, shiftv, zeros2d)
    denT = denp.reshape(NC, N).T
    return _tc2(accp, denT)


# speedup vs baseline: 1.1547x; 1.0699x over previous
import jax
import jax.numpy as jnp
from jax import lax
from jax.experimental import pallas as pl
from jax.experimental.pallas import tpu as pltpu
from jax.experimental.pallas import tpu_sc as plsc

N = 10000
E = 320000
DIM = 128
SLOPE = 0.2

NC = 2
NS = 16
NW = NC * NS
EPW = E // NW
EB = 112
NB = EPW // EB
EB_T = EPW - NB * EB
RPS = 624
RPS_LAST = N - RPS * (NS - 1)


def _tc1_body(h_ref, wt_ref, a8_ref, z_ref, s12_ref, smax_ref):
    i = pl.program_id(0)
    zb = jnp.dot(h_ref[...], wt_ref[...], preferred_element_type=jnp.float32)
    z_ref[...] = zb
    s12 = lax.dot_general(zb, a8_ref[...], (((1,), (1,)), ((), ())),
                          preferred_element_type=jnp.float32)
    s12_ref[...] = s12
    m = jnp.max(s12, axis=0, keepdims=True)

    @pl.when(i == 0)
    def _():
        smax_ref[...] = m

    @pl.when(i > 0)
    def _():
        smax_ref[...] = jnp.maximum(smax_ref[...], m)


def _tc1(h, wt, a8):
    bm = 1000
    return pl.pallas_call(
        _tc1_body,
        grid=(N // bm,),
        in_specs=[
            pl.BlockSpec((bm, DIM), lambda i: (i, 0)),
            pl.BlockSpec((DIM, DIM), lambda i: (0, 0)),
            pl.BlockSpec((8, DIM), lambda i: (0, 0)),
        ],
        out_specs=[
            pl.BlockSpec((bm, DIM), lambda i: (i, 0)),
            pl.BlockSpec((bm, 8), lambda i: (i, 0)),
            pl.BlockSpec((1, 8), lambda i: (0, 0)),
        ],
        out_shape=[
            jax.ShapeDtypeStruct((N, DIM), jnp.float32),
            jax.ShapeDtypeStruct((N, 8), jnp.float32),
            jax.ShapeDtypeStruct((1, 8), jnp.float32),
        ],
    )(h, wt, a8)


def _sc_body(z_hbm, eflat_hbm, s1_hbm, s2_hbm, shift_hbm,
             acc_out, den_out,
             s1_v, s2_v, shift_v,
             srcv0, dstv0, dsc0, exv0, rows0,
             srcv1, dstv1, dsc1, exv1, rows1,
             srcv_t, dsc_t, exv_t,
             acc_sp, den_sp,
             isem0, isem1, gsem0, gsem1, ssem0, ssem1):
    c = lax.axis_index("c")
    s = lax.axis_index("s")
    wid = s * NC + c
    ebase = wid * EPW

    z16 = jnp.zeros((16,), jnp.float32)

    def zrows(b, cc):
        for j in range(DIM // 16):
            rows0[b, pl.ds(j * 16, 16)] = z16
        return cc

    lax.fori_loop(0, EB, zrows, 0)
    for k in range(RPS // EB):
        pltpu.sync_copy(rows0, acc_sp.at[pl.ds(s * RPS + k * EB, EB)])
    rem = RPS - (RPS // EB) * EB
    rem_last = RPS_LAST - (RPS // EB) * EB

    @pl.when(s < NS - 1)
    def _():
        pltpu.sync_copy(rows0.at[pl.ds(0, rem)],
                        acc_sp.at[pl.ds(s * RPS + (RPS // EB) * EB, rem)])

    @pl.when(s == NS - 1)
    def _():
        pltpu.sync_copy(
            rows0.at[pl.ds(0, rem_last)],
            acc_sp.at[pl.ds((NS - 1) * RPS + (RPS // EB) * EB, rem_last)])

    @pl.when(s == 0)
    def _():
        def zloop(i, cc):
            s1_v[pl.ds(i * 16, 16)] = z16
            return cc

        lax.fori_loop(0, N // 16, zloop, 0)
        pltpu.sync_copy(s1_v, den_sp)

    pltpu.sync_copy(s1_hbm, s1_v)
    pltpu.sync_copy(s2_hbm, s2_v)
    pltpu.sync_copy(shift_hbm, shift_v)
    plsc.subcore_barrier()

    shift_vec = shift_v[...]

    slot0 = (srcv0, dstv0, dsc0, exv0, rows0, isem0, gsem0, ssem0)
    slot1 = (srcv1, dstv1, dsc1, exv1, rows1, isem1, gsem1, ssem1)

    def idx_start(u, slot):
        srcv, dstv, _, _, _, isem, _, _ = slot
        base = ebase + u * EB
        pltpu.async_copy(eflat_hbm.at[pl.ds(base, EB)], srcv, isem)
        pltpu.async_copy(eflat_hbm.at[pl.ds(E + base, EB)], dstv, isem)

    def idx_wait(u, slot):
        srcv, dstv, _, _, _, isem, _, _ = slot
        base = ebase + u * EB
        pltpu.make_async_copy(
            eflat_hbm.at[pl.ds(base, EB)], srcv, isem).wait()
        pltpu.make_async_copy(
            eflat_hbm.at[pl.ds(E + base, EB)], dstv, isem).wait()

    def gat_start(slot):
        srcv, _, _, _, rows, _, gsem, _ = slot
        pltpu.async_copy(z_hbm.at[srcv], rows, gsem)

    def gat_wait(slot):
        srcv, _, _, _, rows, _, gsem, _ = slot
        pltpu.make_async_copy(z_hbm.at[srcv], rows, gsem).wait()

    def scat_wait(slot):
        _, _, dsc, exv, rows, _, _, ssem = slot
        pltpu.make_async_copy(rows, acc_sp.at[dsc], ssem).wait()
        pltpu.make_async_copy(exv, den_sp.at[dsc], ssem).wait()

    def section(u, slot, other, *, first=False, do_idx=True,
                do_next_gather=True):
        srcv, dstv, dsc, exv, rows, isem, gsem, ssem = slot
        for i in range(EB // 16):
            sl = pl.ds(i * 16, 16)
            sidx = srcv[sl]
            didx = dstv[sl]
            e = plsc.load_gather(s1_v, [sidx]) + plsc.load_gather(s2_v, [didx])
            e = jnp.where(e >= 0, e, SLOPE * e)
            exv[sl] = jnp.exp(e - shift_vec)
            dsc[sl] = didx
        pltpu.async_copy(exv, den_sp.at[dsc], ssem, add=True)
        gat_wait(slot)
        if do_next_gather:
            if not first:
                scat_wait(other)
            idx_wait(u + 1, other)
            gat_start(other)

        @plsc.parallel_loop(0, EB // 16, unroll=2)
        def _(g):
            ex16 = exv[pl.ds(g * 16, 16)]
            for k in range(16):
                b = g * 16 + k
                sv = jnp.full((16,), ex16[k], jnp.float32)
                for j in range(DIM // 16):
                    rows[b, pl.ds(j * 16, 16)] = rows[b, pl.ds(j * 16, 16)] * sv

        if do_idx:
            idx_start(u + 2, slot)
        pltpu.async_copy(rows, acc_sp.at[dsc], ssem, add=True)

    idx_start(0, slot0)
    idx_start(1, slot1)
    idx_wait(0, slot0)
    gat_start(slot0)
    section(0, slot0, slot1, first=True)

    def pair(g, carry):
        section(2 * g + 1, slot1, slot0)
        section(2 * g + 2, slot0, slot1)
        return carry

    lax.fori_loop(0, (NB - 3) // 2, pair, 0)
    section(NB - 2, slot1, slot0, do_idx=False)
    section(NB - 1, slot0, slot1, do_idx=False, do_next_gather=False)
    scat_wait(slot1)
    scat_wait(slot0)

    tbase = ebase + NB * EB
    rows_t = rows0.at[pl.ds(0, EB_T)]
    pltpu.async_copy(eflat_hbm.at[pl.ds(tbase, EB_T)], srcv_t, isem0)
    pltpu.async_copy(eflat_hbm.at[pl.ds(E + tbase, EB_T)], dsc_t, isem0)
    pltpu.make_async_copy(
        eflat_hbm.at[pl.ds(tbase, EB_T)], srcv_t, isem0).wait()
    pltpu.make_async_copy(
        eflat_hbm.at[pl.ds(E + tbase, EB_T)], dsc_t, isem0).wait()
    pltpu.async_copy(z_hbm.at[srcv_t], rows_t, gsem0)
    for i in range(EB_T // 16):
        sl = pl.ds(i * 16, 16)
        sidx = srcv_t[sl]
        didx = dsc_t[sl]
        e = plsc.load_gather(s1_v, [sidx]) + plsc.load_gather(s2_v, [didx])
        e = jnp.where(e >= 0, e, SLOPE * e)
        exv_t[sl] = jnp.exp(e - shift_vec)
    pltpu.async_copy(exv_t, den_sp.at[dsc_t], ssem0, add=True)
    pltpu.make_async_copy(z_hbm.at[srcv_t], rows_t, gsem0).wait()
    for g in range(EB_T // 16):
        ex16 = exv_t[pl.ds(g * 16, 16)]
        for k in range(16):
            b = g * 16 + k
            sv = jnp.full((16,), ex16[k], jnp.float32)
            for j in range(DIM // 16):
                rows0[b, pl.ds(j * 16, 16)] = rows0[b, pl.ds(j * 16, 16)] * sv
    pltpu.async_copy(rows_t, acc_sp.at[dsc_t], ssem0, add=True)
    pltpu.make_async_copy(exv_t, den_sp.at[dsc_t], ssem0).wait()
    pltpu.make_async_copy(rows_t, acc_sp.at[dsc_t], ssem0).wait()
    plsc.subcore_barrier()

    @pl.when(s < NS - 1)
    def _():
        pltpu.sync_copy(acc_sp.at[pl.ds(s * RPS, RPS)],
                        acc_out.at[pl.ds(c * N + s * RPS, RPS)])

    @pl.when(s == NS - 1)
    def _():
        pltpu.sync_copy(acc_sp.at[pl.ds((NS - 1) * RPS, RPS_LAST)],
                        acc_out.at[pl.ds(c * N + (NS - 1) * RPS, RPS_LAST)])

    @pl.when(s == 0)
    def _():
        pltpu.sync_copy(den_sp, s1_v)
        pltpu.sync_copy(s1_v, den_out.at[pl.ds(c * N, N)])


def _sc_aggregate(z, eflat, s1, s2, shiftv):
    mesh = plsc.VectorSubcoreMesh(core_axis_name="c", subcore_axis_name="s")
    f = pl.kernel(
        _sc_body,
        out_type=[
            jax.ShapeDtypeStruct((NC * N, DIM), jnp.float32),
            jax.ShapeDtypeStruct((NC * N,), jnp.float32),
        ],
        mesh=mesh,
        compiler_params=pltpu.CompilerParams(needs_layout_passes=False),
        scratch_types=(
            [
                pltpu.VMEM((N,), jnp.float32),
                pltpu.VMEM((N,), jnp.float32),
                pltpu.VMEM((16,), jnp.float32),
            ]
            + 2 * [
                pltpu.VMEM((EB,), jnp.int32),
                pltpu.VMEM((EB,), jnp.int32),
                pltpu.VMEM((EB,), jnp.int32),
                pltpu.VMEM((EB,), jnp.float32),
                pltpu.VMEM((EB, DIM), jnp.float32),
            ]
            + [
                pltpu.VMEM((EB_T,), jnp.int32),
                pltpu.VMEM((EB_T,), jnp.int32),
                pltpu.VMEM((EB_T,), jnp.float32),
            ]
            + [
                pltpu.VMEM_SHARED((N, DIM), jnp.float32),
                pltpu.VMEM_SHARED((N,), jnp.float32),
            ]
            + 6 * [pltpu.SemaphoreType.DMA]
        ),
    )
    return f(z, eflat, s1, s2, shiftv)


def _tc2_body(a0_ref, a1_ref, den_ref, out_ref):
    d = den_ref[...]
    dsum = jnp.maximum(d[:, 0:1] + d[:, 1:2], 1e-9)
    hout = (a0_ref[...] + a1_ref[...]) / dsum
    out_ref[...] = jnp.where(hout > 0, hout, jnp.exp(hout) - 1.0)


def _tc2(accp, denT):
    bm = 1000
    g = N // bm
    return pl.pallas_call(
        _tc2_body,
        grid=(g,),
        in_specs=[
            pl.BlockSpec((bm, DIM), lambda i: (i, 0)),
            pl.BlockSpec((bm, DIM), lambda i, g=g: (i + g, 0)),
            pl.BlockSpec((bm, 2), lambda i: (i, 0)),
        ],
        out_specs=pl.BlockSpec((bm, DIM), lambda i: (i, 0)),
        out_shape=jax.ShapeDtypeStruct((N, DIM), jnp.float32),
    )(accp, accp, denT)


@jax.jit
def kernel(h, edge_index, W, a):
    wt = W.T
    a8 = jnp.zeros((8, DIM), jnp.float32)
    a8 = a8.at[0].set(a[0, :DIM]).at[1].set(a[0, DIM:])
    z, s12, smax = _tc1(h, wt, a8)
    s1 = s12[:, 0]
    s2 = s12[:, 1]
    shift = jnp.maximum(smax[0, 0] + smax[0, 1], 0.0)
    shiftv = jnp.full((16,), shift, jnp.float32)
    eflat = edge_index.reshape(2 * E)
    accp, denp = _sc_aggregate(z, eflat, s1, s2, shiftv)
    denT = denp.reshape(NC, N).T
    return _tc2(accp, denT)
